# baseline (speedup 1.0000x reference)
import functools

import jax
import jax.numpy as jnp
from jax import lax
from jax.experimental import pallas as pl
from jax.experimental.pallas import tpu as pltpu
from jax.experimental.pallas import tpu_sc as plsc

N_TOKENS = 32768
D_MODEL = 768
N_EXPERTS = 64
N_ACTIVE = 8
BLOCK_ROWS = 4096

_NUM_SC = 2
_NUM_SUBCORES = 16
_NW = _NUM_SC * _NUM_SUBCORES
_ROWS_PER_W = N_TOKENS // _NW
_LANES = 16


def _probs_block(x_ref, w_ref, probs_ref):
    x = x_ref[...]
    w = w_ref[...]
    lt = jax.lax.dot_general(
        w, x, (((1,), (1,)), ((), ())), preferred_element_type=jnp.float32
    )
    m = jnp.max(lt, axis=0, keepdims=True)
    et = jnp.exp(lt - m)
    s = jnp.sum(et, axis=0, keepdims=True)
    probs_ref[...] = (et / s).T


def _tc_probs(x, W):
    n = x.shape[0]
    return pl.pallas_call(
        _probs_block,
        grid=(n // BLOCK_ROWS,),
        in_specs=[
            pl.BlockSpec((BLOCK_ROWS, D_MODEL), lambda i: (i, 0)),
            pl.BlockSpec((N_EXPERTS, D_MODEL), lambda i: (0, 0)),
        ],
        out_specs=pl.BlockSpec((BLOCK_ROWS, N_EXPERTS), lambda i: (i, 0)),
        out_shape=jax.ShapeDtypeStruct((n, N_EXPERTS), jnp.float32),
    )(x, W)


def _merge_top8(ak, av, bk, bv, lo8):
    rbk = lax.rev(bk, (0,))
    rbv = lax.rev(bv, (0,))
    mk = jnp.where(lo8, ak, rbk)
    mv = jnp.where(lo8, av, rbv)
    return plsc.sort_key_val(mk, mv, descending=True)


_CHUNK = 256
_N_CHUNKS = _ROWS_PER_W // _CHUNK


def _sc_topk_body(probs_hbm, vals_hbm, idx_hbm, pbuf, vbuf, ibuf):
    wid = lax.axis_index("s") * _NUM_SC + lax.axis_index("c")
    base = wid * _ROWS_PER_W

    lane = lax.iota(jnp.int32, _LANES)
    lo8 = lane < N_ACTIVE

    for chunk in range(_N_CHUNKS):
        cbase = base + chunk * _CHUNK
        pltpu.sync_copy(probs_hbm.at[pl.ds(cbase, _CHUNK)], pbuf)

        @plsc.parallel_loop(0, _CHUNK, unroll=4)
        def _row(r):
            ks = []
            vs = []
            for c in range(N_EXPERTS // _LANES):
                k = pbuf[r, pl.ds(c * _LANES, _LANES)]
                sk, sv = plsc.sort_key_val(
                    k, lane + c * _LANES, descending=True
                )
                ks.append(sk)
                vs.append(sv)
            k01, v01 = _merge_top8(ks[0], vs[0], ks[1], vs[1], lo8)
            k23, v23 = _merge_top8(ks[2], vs[2], ks[3], vs[3], lo8)
            k8, v8 = _merge_top8(k01, v01, k23, v23, lo8)
            s8 = jnp.sum(jnp.where(lo8, k8, 0.0))
            vals = k8 / (s8 + 1e-6)
            off = pl.multiple_of(r * N_ACTIVE, 8)
            plsc.store_compressed(vbuf.at[pl.ds(off, _LANES)], vals, mask=lo8)
            plsc.store_compressed(ibuf.at[pl.ds(off, _LANES)], v8, mask=lo8)

        nout = _CHUNK * N_ACTIVE
        pltpu.sync_copy(
            vbuf.at[pl.ds(0, nout)],
            vals_hbm.at[pl.ds(cbase * N_ACTIVE, nout)],
        )
        pltpu.sync_copy(
            ibuf.at[pl.ds(0, nout)],
            idx_hbm.at[pl.ds(cbase * N_ACTIVE, nout)],
        )


_sc_topk = functools.partial(
    pl.kernel,
    out_type=[
        jax.ShapeDtypeStruct((N_TOKENS * N_ACTIVE,), jnp.float32),
        jax.ShapeDtypeStruct((N_TOKENS * N_ACTIVE,), jnp.int32),
    ],
    mesh=plsc.VectorSubcoreMesh(core_axis_name="c", subcore_axis_name="s"),
    compiler_params=pltpu.CompilerParams(needs_layout_passes=False),
    scratch_types=[
        pltpu.VMEM((_CHUNK, N_EXPERTS), jnp.float32),
        pltpu.VMEM((_CHUNK * N_ACTIVE + 8,), jnp.float32),
        pltpu.VMEM((_CHUNK * N_ACTIVE + 8,), jnp.int32),
    ],
)(_sc_topk_body)


@jax.jit
def kernel(x, W):
    probs = _tc_probs(x, W)
    vals_flat, idx_flat = _sc_topk(probs)
    vals = vals_flat.reshape(N_TOKENS, N_ACTIVE)
    idx = idx_flat.reshape(N_TOKENS, N_ACTIVE)
    return (vals, idx, probs)


# speedup vs baseline: 1.0167x; 1.0167x over previous
import functools

import jax
import jax.numpy as jnp
from jax import lax
from jax.experimental import pallas as pl
from jax.experimental.pallas import tpu as pltpu
from jax.experimental.pallas import tpu_sc as plsc

N_TOKENS = 32768
D_MODEL = 768
N_EXPERTS = 64
N_ACTIVE = 8
BLOCK_ROWS = 4096

_NUM_SC = 2
_NUM_SUBCORES = 16
_NW = _NUM_SC * _NUM_SUBCORES
_ROWS_PER_W = N_TOKENS // _NW
_LANES = 16


def _probs_block(x_ref, w_ref, probs_ref):
    x = x_ref[...]
    w = w_ref[...]
    lt = jax.lax.dot_general(
        w, x, (((1,), (1,)), ((), ())), preferred_element_type=jnp.float32
    )
    m = jnp.max(lt, axis=0, keepdims=True)
    et = jnp.exp(lt - m)
    s = jnp.sum(et, axis=0, keepdims=True)
    probs_ref[...] = (et / s).T


def _tc_probs(x, W):
    n = x.shape[0]
    return pl.pallas_call(
        _probs_block,
        grid=(n // BLOCK_ROWS,),
        in_specs=[
            pl.BlockSpec((BLOCK_ROWS, D_MODEL), lambda i: (i, 0)),
            pl.BlockSpec((N_EXPERTS, D_MODEL), lambda i: (0, 0)),
        ],
        out_specs=pl.BlockSpec((BLOCK_ROWS, N_EXPERTS), lambda i: (i, 0)),
        out_shape=jax.ShapeDtypeStruct((n, N_EXPERTS), jnp.float32),
    )(x, W)


def _merge_top8(ak, av, bk, bv, lo8, descending):
    mk = jnp.where(lo8, ak, bk)
    mv = jnp.where(lo8, av, bv)
    return plsc.sort_key_val(mk, mv, descending=descending)


_CHUNK = 256
_N_CHUNKS = _ROWS_PER_W // _CHUNK


def _sc_topk_body(probs_hbm, vals_hbm, idx_hbm, pbuf, vbuf, ibuf):
    wid = lax.axis_index("s") * _NUM_SC + lax.axis_index("c")
    base = wid * _ROWS_PER_W

    lane = lax.iota(jnp.int32, _LANES)
    lo8 = lane < N_ACTIVE

    for chunk in range(_N_CHUNKS):
        cbase = base + chunk * _CHUNK
        pltpu.sync_copy(probs_hbm.at[pl.ds(cbase, _CHUNK)], pbuf)

        @plsc.parallel_loop(0, _CHUNK, unroll=2)
        def _row(r):
            ks = []
            vs = []
            for c in range(N_EXPERTS // _LANES):
                k = pbuf[r, pl.ds(c * _LANES, _LANES)]
                sk, sv = plsc.sort_key_val(
                    k, lane + c * _LANES, descending=(c % 2 == 0)
                )
                ks.append(sk)
                vs.append(sv)
            k01, v01 = _merge_top8(
                ks[0], vs[0], ks[1], vs[1], lo8, descending=True
            )
            k23, v23 = _merge_top8(
                ks[2], vs[2], ks[3], vs[3], lo8, descending=False
            )
            k8, v8 = _merge_top8(k01, v01, k23, v23, lo8, descending=True)
            s8 = jnp.sum(jnp.where(lo8, k8, 0.0))
            vals = k8 / (s8 + 1e-6)
            off = pl.multiple_of(r * N_ACTIVE, 8)
            plsc.store_compressed(vbuf.at[pl.ds(off, _LANES)], vals, mask=lo8)
            plsc.store_compressed(ibuf.at[pl.ds(off, _LANES)], v8, mask=lo8)

        nout = _CHUNK * N_ACTIVE
        pltpu.sync_copy(
            vbuf.at[pl.ds(0, nout)],
            vals_hbm.at[pl.ds(cbase * N_ACTIVE, nout)],
        )
        pltpu.sync_copy(
            ibuf.at[pl.ds(0, nout)],
            idx_hbm.at[pl.ds(cbase * N_ACTIVE, nout)],
        )


_sc_topk = functools.partial(
    pl.kernel,
    out_type=[
        jax.ShapeDtypeStruct((N_TOKENS * N_ACTIVE,), jnp.float32),
        jax.ShapeDtypeStruct((N_TOKENS * N_ACTIVE,), jnp.int32),
    ],
    mesh=plsc.VectorSubcoreMesh(core_axis_name="c", subcore_axis_name="s"),
    compiler_params=pltpu.CompilerParams(needs_layout_passes=False),
    scratch_types=[
        pltpu.VMEM((_CHUNK, N_EXPERTS), jnp.float32),
        pltpu.VMEM((_CHUNK * N_ACTIVE + 8,), jnp.float32),
        pltpu.VMEM((_CHUNK * N_ACTIVE + 8,), jnp.int32),
    ],
)(_sc_topk_body)


@jax.jit
def kernel(x, W):
    probs = _tc_probs(x, W)
    vals_flat, idx_flat = _sc_topk(probs)
    vals = vals_flat.reshape(N_TOKENS, N_ACTIVE)
    idx = idx_flat.reshape(N_TOKENS, N_ACTIVE)
    return (vals, idx, probs)


# speedup vs baseline: 1.0376x; 1.0206x over previous
import functools

import jax
import jax.numpy as jnp
from jax import lax
from jax.experimental import pallas as pl
from jax.experimental.pallas import tpu as pltpu
from jax.experimental.pallas import tpu_sc as plsc

N_TOKENS = 32768
D_MODEL = 768
N_EXPERTS = 64
N_ACTIVE = 8
BLOCK_ROWS = 4096

_NUM_SC = 2
_NUM_SUBCORES = 16
_NW = _NUM_SC * _NUM_SUBCORES
_ROWS_PER_W = N_TOKENS // _NW
_LANES = 16


def _probs_block(x_ref, w_ref, probs_ref):
    x = x_ref[...]
    w = w_ref[...]
    lt = jax.lax.dot_general(
        w, x, (((1,), (1,)), ((), ())), preferred_element_type=jnp.float32
    )
    m = jnp.max(lt, axis=0, keepdims=True)
    et = jnp.exp(lt - m)
    s = jnp.sum(et, axis=0, keepdims=True)
    probs_ref[...] = (et / s).T


def _tc_probs(x, W):
    n = x.shape[0]
    return pl.pallas_call(
        _probs_block,
        grid=(n // BLOCK_ROWS,),
        in_specs=[
            pl.BlockSpec((BLOCK_ROWS, D_MODEL), lambda i: (i, 0)),
            pl.BlockSpec((N_EXPERTS, D_MODEL), lambda i: (0, 0)),
        ],
        out_specs=pl.BlockSpec((BLOCK_ROWS, N_EXPERTS), lambda i: (i, 0)),
        out_shape=jax.ShapeDtypeStruct((n, N_EXPERTS), jnp.float32),
    )(x, W)


def _merge_top8(ak, av, bk, bv, lo8, descending):
    mk = jnp.where(lo8, ak, bk)
    mv = jnp.where(lo8, av, bv)
    return plsc.sort_key_val(mk, mv, descending=descending)


_CHUNK = 512
_N_CHUNKS = _ROWS_PER_W // _CHUNK


def _sc_topk_body(probs_hbm, vals_hbm, idx_hbm, pbuf, vbuf, ibuf):
    wid = lax.axis_index("s") * _NUM_SC + lax.axis_index("c")
    base = wid * _ROWS_PER_W

    lane = lax.iota(jnp.int32, _LANES)
    lo8 = lane < N_ACTIVE

    for chunk in range(_N_CHUNKS):
        cbase = base + chunk * _CHUNK
        pltpu.sync_copy(probs_hbm.at[pl.ds(cbase, _CHUNK)], pbuf)

        @plsc.parallel_loop(0, _CHUNK)
        def _row(r):
            ks = []
            vs = []
            for c in range(N_EXPERTS // _LANES):
                k = pbuf[r, pl.ds(c * _LANES, _LANES)]
                sk, sv = plsc.sort_key_val(
                    k, lane + c * _LANES, descending=(c % 2 == 0)
                )
                ks.append(sk)
                vs.append(sv)
            k01, v01 = _merge_top8(
                ks[0], vs[0], ks[1], vs[1], lo8, descending=True
            )
            k23, v23 = _merge_top8(
                ks[2], vs[2], ks[3], vs[3], lo8, descending=False
            )
            k8, v8 = _merge_top8(k01, v01, k23, v23, lo8, descending=True)
            s8 = jnp.sum(jnp.where(lo8, k8, 0.0))
            vals = k8 / (s8 + 1e-6)
            off = pl.multiple_of(r * N_ACTIVE, 8)
            plsc.store_compressed(vbuf.at[pl.ds(off, _LANES)], vals, mask=lo8)
            plsc.store_compressed(ibuf.at[pl.ds(off, _LANES)], v8, mask=lo8)

        nout = _CHUNK * N_ACTIVE
        pltpu.sync_copy(
            vbuf.at[pl.ds(0, nout)],
            vals_hbm.at[pl.ds(cbase * N_ACTIVE, nout)],
        )
        pltpu.sync_copy(
            ibuf.at[pl.ds(0, nout)],
            idx_hbm.at[pl.ds(cbase * N_ACTIVE, nout)],
        )


_sc_topk = functools.partial(
    pl.kernel,
    out_type=[
        jax.ShapeDtypeStruct((N_TOKENS * N_ACTIVE,), jnp.float32),
        jax.ShapeDtypeStruct((N_TOKENS * N_ACTIVE,), jnp.int32),
    ],
    mesh=plsc.VectorSubcoreMesh(core_axis_name="c", subcore_axis_name="s"),
    compiler_params=pltpu.CompilerParams(needs_layout_passes=False),
    scratch_types=[
        pltpu.VMEM((_CHUNK, N_EXPERTS), jnp.float32),
        pltpu.VMEM((_CHUNK * N_ACTIVE + 8,), jnp.float32),
        pltpu.VMEM((_CHUNK * N_ACTIVE + 8,), jnp.int32),
    ],
)(_sc_topk_body)


@jax.jit
def kernel(x, W):
    probs = _tc_probs(x, W)
    vals_flat, idx_flat = _sc_topk(probs)
    vals = vals_flat.reshape(N_TOKENS, N_ACTIVE)
    idx = idx_flat.reshape(N_TOKENS, N_ACTIVE)
    return (vals, idx, probs)
